# baseline (speedup 1.0000x reference)
import functools
import math

import numpy as np

import jax
import jax.numpy as jnp
from jax.experimental import pallas as pl
from jax.experimental.pallas import tpu as pltpu

BF16 = jnp.bfloat16


def _ru(x, m):
    return ((x + m - 1) // m) * m


def _pad2d(a, rows, cols):
    if a.shape[0] == rows and a.shape[1] == cols:
        return a
    return jnp.pad(a, ((0, rows - a.shape[0]), (0, cols - a.shape[1])))


def _mm_body(a_ref, b_ref, s_ref, t_ref, *rest, relu, has_res):
    if has_res:
        r_ref, o_ref = rest
    else:
        (o_ref,) = rest
    acc = jnp.dot(a_ref[...], b_ref[...], preferred_element_type=jnp.float32)
    acc = acc * s_ref[...] + t_ref[...]
    if has_res:
        acc = acc + r_ref[...].astype(jnp.float32)
    if relu:
        acc = jnp.maximum(acc, 0.0)
    o_ref[...] = acc.astype(o_ref.dtype)


def _mm_bn(a, w, s, t, *, relu, residual=None, out_dtype=BF16):
    M, K = a.shape
    N = w.shape[1]
    a = a.astype(BF16)
    w = w.astype(BF16)

    bm = min(512, _ru(M, 8))
    bn = min(256, _ru(N, 128))
    Kp = _ru(K, 128)
    if M // bm < 2 and bm > 8:
        bm = max(8, _ru(bm // 2, 8))
    Mp, Np = _ru(M, bm), _ru(N, bn)

    args = [_pad2d(a, Mp, Kp), _pad2d(w, Kp, Np),
            _pad2d(s, 1, Np), _pad2d(t, 1, Np)]
    has_res = residual is not None
    if has_res:
        args.append(_pad2d(residual.astype(BF16), Mp, Np))

    in_specs = [pl.BlockSpec((bm, Kp), lambda i, j: (i, 0)),
                pl.BlockSpec((Kp, bn), lambda i, j: (0, j)),
                pl.BlockSpec((1, bn), lambda i, j: (0, j)),
                pl.BlockSpec((1, bn), lambda i, j: (0, j))]
    if has_res:
        in_specs.append(pl.BlockSpec((bm, bn), lambda i, j: (i, j)))

    out = pl.pallas_call(
        functools.partial(_mm_body, relu=relu, has_res=has_res),
        out_shape=jax.ShapeDtypeStruct((Mp, Np), out_dtype),
        grid=(Mp // bm, Np // bn),
        in_specs=in_specs,
        out_specs=pl.BlockSpec((bm, bn), lambda i, j: (i, j)),
        compiler_params=pltpu.CompilerParams(
            dimension_semantics=("parallel", "parallel"),
            vmem_limit_bytes=64 * 1024 * 1024),
    )(*args)
    return out[:M, :N]


def _conv1x1(x, w, s, t, *, stride=1, relu, residual=None):
    if stride > 1:
        x = x[:, ::stride, ::stride, :]
    B, H, W, C = x.shape
    N = w.shape[-1]
    res = None if residual is None else residual.reshape(B * H * W, N)
    out = _mm_bn(x.reshape(B * H * W, C), w, s, t, relu=relu, residual=res)
    return out.reshape(B, H, W, N)


def _conv_knxn(x, w, s, t, *, stride, padding, relu):
    B, H, W, C = x.shape
    KH, KW, _, N = w.shape
    xp = jnp.pad(x, ((0, 0), (padding, padding), (padding, padding), (0, 0)))
    OH = (H + 2 * padding - KH) // stride + 1
    OW = (W + 2 * padding - KW) // stride + 1
    taps = [xp[:, kh:kh + stride * (OH - 1) + 1:stride,
               kw:kw + stride * (OW - 1) + 1:stride, :]
            for kh in range(KH) for kw in range(KW)]
    patches = jnp.concatenate(taps, axis=-1).reshape(B * OH * OW, KH * KW * C)
    out = _mm_bn(patches, w.reshape(KH * KW * C, N), s, t, relu=relu)
    return out.reshape(B, OH, OW, N)


def _maxpool_3x3_s2(x):
    init = jnp.array(-jnp.inf, dtype=x.dtype)
    return jax.lax.reduce_window(
        x, init, jax.lax.max,
        window_dimensions=(1, 3, 3, 1), window_strides=(1, 2, 2, 1),
        padding=((0, 0), (1, 1), (1, 1), (0, 0)))


def _stack_body(x0_ref, m_ref, w1_ref, s1_ref, t1_ref,
                w2_ref, s2_ref, t2_ref, w3_ref, s3_ref, t3_ref,
                o_ref, h_ref, *, g, wrow, rows):
    blk = pl.program_id(1)

    @pl.when(blk == 0)
    def _():
        o_ref[...] = x0_ref[...]

    act = o_ref[0]

    h1 = jnp.dot(act, w1_ref[0], preferred_element_type=jnp.float32)
    h1 = jnp.maximum(h1 * s1_ref[0] + t1_ref[0], 0.0) * m_ref[...]
    h_ref[...] = h1.astype(BF16)

    shifts = [h_ref[g + dh * wrow + dw:g + dh * wrow + dw + rows, :]
              for dh in (-1, 0, 1) for dw in (-1, 0, 1)]
    wide = jnp.concatenate(shifts, axis=1)
    h2 = jnp.dot(wide, w2_ref[0], preferred_element_type=jnp.float32)
    h2 = jnp.maximum(h2 * s2_ref[0] + t2_ref[0], 0.0).astype(BF16)

    o3 = jnp.dot(h2, w3_ref[0], preferred_element_type=jnp.float32)
    o3 = o3 * s3_ref[0] + t3_ref[0] + act[g:g + rows, :].astype(jnp.float32)
    o_ref[0, g:g + rows, :] = jnp.maximum(o3, 0.0).astype(BF16)


def _fold_groups(x, G, n_pad):
    B, H, W, C = x.shape
    n = (H + 2) * (W + 2)
    xp = jnp.pad(x, ((0, 0), (1, 1), (1, 1), (0, 0))).reshape(B, n, C)
    xp = jnp.pad(xp, ((0, 0), (G, n_pad - n), (0, 0)))
    xp = xp.reshape(2, (B // 2) * (G + n_pad), C)
    return jnp.pad(xp, ((0, 0), (0, G), (0, 0)))


def _unfold_groups(o, B, H, W, G, n_pad):
    Hp, Wp = H + 2, W + 2
    seg = G + n_pad
    body = o[:, :(B // 2) * seg, :].reshape(B, seg, -1)[:, G:G + Hp * Wp, :]
    return body.reshape(B, Hp, Wp, -1)[:, 1:H + 1, 1:W + 1, :]


def _row_mask(H, W, G, n_pad, per_group):
    Hp, Wp = H + 2, W + 2
    m = np.zeros((Hp, Wp), np.float32)
    m[1:H + 1, 1:W + 1] = 1.0
    seg = np.concatenate([np.zeros(G, np.float32), m.reshape(-1),
                          np.zeros(n_pad - Hp * Wp, np.float32)])
    full = np.concatenate([np.tile(seg, per_group), np.zeros(G, np.float32)])
    return jnp.asarray(full.reshape(-1, 1))


def _bottleneck_stack(x, w1, s1, t1, w2, s2, t2, w3, s3, t3):
    B, H, W, C = x.shape
    NB, _, Pp = w1.shape
    Wp = W + 2
    n = (H + 2) * Wp
    n_pad = _ru(n, 8)
    G = _ru(Wp + 1, 8)
    Rg = (B // 2) * (G + n_pad) + G
    rows = Rg - 2 * G

    x0 = _fold_groups(x.astype(BF16), G, n_pad)
    mask = _row_mask(H, W, G, n_pad, B // 2)
    w2k = w2.reshape(NB, 9 * Pp, Pp)

    out = pl.pallas_call(
        functools.partial(_stack_body, g=G, wrow=Wp, rows=rows),
        out_shape=jax.ShapeDtypeStruct((2, Rg, C), BF16),
        grid_spec=pltpu.PrefetchScalarGridSpec(
            num_scalar_prefetch=0,
            grid=(2, NB),
            in_specs=[
                pl.BlockSpec((1, Rg, C), lambda g, b: (g, 0, 0)),
                pl.BlockSpec((Rg, 1), lambda g, b: (0, 0)),
                pl.BlockSpec((1, C, Pp), lambda g, b: (b, 0, 0)),
                pl.BlockSpec((1, 1, Pp), lambda g, b: (b, 0, 0)),
                pl.BlockSpec((1, 1, Pp), lambda g, b: (b, 0, 0)),
                pl.BlockSpec((1, 9 * Pp, Pp), lambda g, b: (b, 0, 0)),
                pl.BlockSpec((1, 1, Pp), lambda g, b: (b, 0, 0)),
                pl.BlockSpec((1, 1, Pp), lambda g, b: (b, 0, 0)),
                pl.BlockSpec((1, Pp, C), lambda g, b: (b, 0, 0)),
                pl.BlockSpec((1, 1, C), lambda g, b: (b, 0, 0)),
                pl.BlockSpec((1, 1, C), lambda g, b: (b, 0, 0)),
            ],
            out_specs=pl.BlockSpec((1, Rg, C), lambda g, b: (g, 0, 0)),
            scratch_shapes=[pltpu.VMEM((Rg, Pp), BF16)]),
        compiler_params=pltpu.CompilerParams(
            dimension_semantics=("parallel", "arbitrary"),
            vmem_limit_bytes=64 * 1024 * 1024),
    )(x0, mask, w1, s1, t1, w2k, s2, t2, w3, s3, t3)

    return _unfold_groups(out, B, H, W, G, n_pad)


def _b0_body(x_ref, m_ref, w1_ref, s1_ref, t1_ref, w2_ref, s2_ref, t2_ref,
             w3_ref, s3_ref, t3_ref, wd_ref, sd_ref, td_ref, o_ref,
             *, g, bm, nph, cin, pp, taps):
    base = pl.program_id(1) * bm
    xw = x_ref[0, pl.ds(base, bm + 2 * g), :]
    mw = m_ref[pl.ds(base, bm + 2 * g), :]

    hs = []
    for p in range(nph):
        hp = jnp.dot(xw[:, p * cin:(p + 1) * cin], w1_ref[...],
                     preferred_element_type=jnp.float32)
        hs.append((jnp.maximum(hp * s1_ref[...] + t1_ref[...], 0.0)
                   * mw).astype(BF16))

    wide = jnp.concatenate(
        [hs[b][g + s:g + s + bm, :] for s, b in taps], axis=1)
    h2 = jnp.dot(wide, w2_ref[...], preferred_element_type=jnp.float32)
    h2 = jnp.maximum(h2 * s2_ref[...] + t2_ref[...], 0.0).astype(BF16)

    o3 = jnp.dot(h2, w3_ref[...], preferred_element_type=jnp.float32)
    o3 = o3 * s3_ref[...] + t3_ref[...]
    idn = jnp.dot(xw[g:g + bm, :cin], wd_ref[...],
                  preferred_element_type=jnp.float32)
    o3 = o3 + idn * sd_ref[...] + td_ref[...]
    o_ref[0] = jnp.maximum(o3, 0.0).astype(BF16)


def _pad_last(a, n):
    if a.shape[-1] == n:
        return a
    return jnp.pad(a, [(0, 0)] * (a.ndim - 1) + [(0, n - a.shape[-1])])


def _b0_fused(x, p, stride):
    B, H, W, C = x.shape
    if stride == 2:
        H, W = H // 2, W // 2
        x = x.reshape(B, H, 2, W, 2, C).transpose(0, 1, 3, 2, 4, 5)
        x = x.reshape(B, H, W, 4 * C)
        nph = 4
    else:
        nph = 1
    P0 = p["w1"].shape[1]
    C2 = p["w3"].shape[1]
    P = max(P0, 128)
    Cp = max(C, 128)
    x = _pad_last(x, nph * Cp) if C < 128 else x

    Wp = W + 2
    n = (H + 2) * Wp
    n_pad = _ru(n, 8)
    G = _ru(Wp + 1, 8)
    Rg = (B // 2) * (G + n_pad) + G
    T = -(-Rg // 1536)
    bm = _ru(-(-Rg // T), 8)
    Rgp = bm * T

    x0 = _fold_groups(x.astype(BF16), G, n_pad)
    x0 = jnp.pad(x0, ((0, 0), (0, Rgp + 2 * G - Rg), (0, 0)))
    mask = _row_mask(H, W, G, n_pad, B // 2)
    mask = jnp.pad(mask, ((0, Rgp + 2 * G - Rg), (0, 0)))

    w1 = _pad2d(p["w1"].astype(BF16), Cp, P)
    s1, t1 = _pad2d(p["s1"], 1, P), _pad2d(p["t1"], 1, P)
    w2 = p["w2"].reshape(3, 3, P0, P0).reshape(9, P0, P0)
    w2 = jnp.pad(w2, ((0, 0), (0, P - P0), (0, P - P0)))
    w2 = w2.reshape(9 * P, P).astype(BF16)
    s2, t2 = _pad2d(p["s2"], 1, P), _pad2d(p["t2"], 1, P)
    w3 = _pad2d(p["w3"].astype(BF16), P, C2)
    wd = _pad2d(p["wd"].astype(BF16), Cp, C2)

    if nph == 1:
        taps = [(dh * Wp + dw, 0) for dh in (-1, 0, 1) for dw in (-1, 0, 1)]
    else:
        def phm(k):
            return (1, -1) if k == -1 else (0, 0) if k == 0 else (1, 0)
        taps = []
        for kh in (-1, 0, 1):
            ph, di = phm(kh)
            for kw in (-1, 0, 1):
                pw, dj = phm(kw)
                taps.append((di * Wp + dj, ph * 2 + pw))

    Cx = x0.shape[-1]
    out = pl.pallas_call(
        functools.partial(_b0_body, g=G, bm=bm, nph=nph, cin=Cp, pp=P,
                          taps=taps),
        out_shape=jax.ShapeDtypeStruct((2, Rgp, C2), BF16),
        grid=(2, T),
        in_specs=[pl.BlockSpec((1, Rgp + 2 * G, Cx), lambda g, i: (g, 0, 0)),
                  pl.BlockSpec((Rgp + 2 * G, 1), lambda g, i: (0, 0)),
                  pl.BlockSpec((Cp, P), lambda g, i: (0, 0)),
                  pl.BlockSpec((1, P), lambda g, i: (0, 0)),
                  pl.BlockSpec((1, P), lambda g, i: (0, 0)),
                  pl.BlockSpec((9 * P, P), lambda g, i: (0, 0)),
                  pl.BlockSpec((1, P), lambda g, i: (0, 0)),
                  pl.BlockSpec((1, P), lambda g, i: (0, 0)),
                  pl.BlockSpec((P, C2), lambda g, i: (0, 0)),
                  pl.BlockSpec((1, C2), lambda g, i: (0, 0)),
                  pl.BlockSpec((1, C2), lambda g, i: (0, 0)),
                  pl.BlockSpec((Cp, C2), lambda g, i: (0, 0)),
                  pl.BlockSpec((1, C2), lambda g, i: (0, 0)),
                  pl.BlockSpec((1, C2), lambda g, i: (0, 0))],
        out_specs=pl.BlockSpec((1, bm, C2), lambda g, i: (g, i, 0)),
        compiler_params=pltpu.CompilerParams(
            dimension_semantics=("parallel", "parallel"),
            vmem_limit_bytes=64 * 1024 * 1024),
    )(x0, mask, w1, s1, t1, w2, s2, t2, w3, p["s3"], p["t3"],
      wd, p["sd"], p["td"])

    seg = G + n_pad
    body = out.reshape(2, Rgp, C2)[:, :(B // 2) * seg, :].reshape(B, seg, C2)
    body = body[:, :n, :].reshape(B, H + 2, Wp, C2)
    return body[:, 1:H + 1, 1:W + 1, :]


def _stem_body(x_ref, m_ref, w_ref, s_ref, t_ref, o_ref, *, wrow, bm):
    base = pl.program_id(1) * bm
    win = x_ref[0, pl.ds(base, bm + 352), :]
    shifts = [dh * wrow + dw for dh in range(4) for dw in range(4)]
    wide = jnp.concatenate([win[s:s + bm + 144, :] for s in shifts], axis=1)
    h = jnp.dot(wide, w_ref[...], preferred_element_type=jnp.float32)
    h = jnp.maximum(h * s_ref[...] + t_ref[...], 0.0)
    mw = m_ref[pl.ds(base, bm + 144), :]
    h = h * mw - (1.0 - mw) * 1e9
    m1 = jnp.maximum(jnp.maximum(h[3:3 + bm + 136, :], h[4:4 + bm + 136, :]),
                     h[5:5 + bm + 136, :])
    m2 = jnp.maximum(jnp.maximum(m1[1:1 + bm, :], m1[68:68 + bm, :]),
                     m1[135:135 + bm, :])
    o_ref[0] = m2.astype(BF16)


def _stem_conv(x_nchw, w, s, t):
    B = x_nchw.shape[0]
    x = x_nchw.astype(BF16)
    x = jnp.pad(x, ((0, 0), (0, 1), (3, 3), (3, 3)))
    x = x.reshape(B, 4, 67, 2, 67, 2).transpose(0, 2, 4, 3, 5, 1)
    x = x.reshape(B, 67, 67, 16)
    n, n_pad, bm = 67 * 67, 4608, 2304
    rows = (B // 2) * n_pad
    x = jnp.pad(x.reshape(B, n, 16), ((0, 0), (0, n_pad - n), (0, 0)))
    x = x.reshape(2, rows, 16)
    x = jnp.pad(x, ((0, 0), (72, 352), (0, 0)))

    mrow = np.zeros((67, 67), np.float32)
    mrow[:64, :64] = 1.0
    mseg = np.concatenate([mrow.reshape(-1), np.zeros(n_pad - n, np.float32)])
    mfull = np.concatenate([np.zeros(72, np.float32),
                            np.tile(mseg, B // 2),
                            np.zeros(144, np.float32)])
    mask = jnp.asarray(mfull.reshape(-1, 1))

    wp = jnp.pad(w, ((0, 1), (0, 1), (0, 1), (0, 0)))
    wk = wp.reshape(4, 2, 4, 2, 4, 64).transpose(0, 2, 1, 3, 4, 5)
    wk = wk.reshape(256, 64).astype(BF16)
    wk = jnp.pad(wk, ((0, 0), (0, 64)))
    sp = _pad2d(s, 1, 128)
    tp = _pad2d(t, 1, 128)

    out = pl.pallas_call(
        functools.partial(_stem_body, wrow=67, bm=bm),
        out_shape=jax.ShapeDtypeStruct((2, rows, 128), BF16),
        grid=(2, rows // bm),
        in_specs=[pl.BlockSpec((1, rows + 424, 16), lambda g, i: (g, 0, 0)),
                  pl.BlockSpec((rows + 216, 1), lambda g, i: (0, 0)),
                  pl.BlockSpec((256, 128), lambda g, i: (0, 0)),
                  pl.BlockSpec((1, 128), lambda g, i: (0, 0)),
                  pl.BlockSpec((1, 128), lambda g, i: (0, 0))],
        out_specs=pl.BlockSpec((1, bm, 128), lambda g, i: (g, i, 0)),
        compiler_params=pltpu.CompilerParams(
            dimension_semantics=("parallel", "parallel"),
            vmem_limit_bytes=64 * 1024 * 1024),
    )(x, mask, wk, sp, tp)

    out = out.reshape(B, n_pad, 128)[:, :n, :].reshape(B, 67, 67, 128)
    return out[:, 0:64:2, 0:64:2, :]


def _head_body(f_ref, y_ref, wr_ref, br_ref, wu_ref, bu_ref,
               wfr_ref, wfu_ref, bf_ref, o_ref):
    pooled = jnp.mean(f_ref[...].astype(jnp.float32), axis=1)
    r = jnp.dot(pooled.astype(BF16), wr_ref[...],
                preferred_element_type=jnp.float32) + br_ref[...]
    u = jnp.dot(y_ref[...], wu_ref[...],
                preferred_element_type=jnp.float32) + bu_ref[...]
    z = (jnp.dot(r.astype(BF16), wfr_ref[...],
                 preferred_element_type=jnp.float32)
         + jnp.dot(u.astype(BF16), wfu_ref[...],
                   preferred_element_type=jnp.float32)
         + bf_ref[...])
    o_ref[...] = z


def _head(feat, y2, wr, br, wu, bu, wfr, wfu, bf):
    B = feat.shape[0]
    args = (feat, y2, wr, br, wu, bu, wfr, wfu, bf)
    in_specs = [pl.BlockSpec(a.shape, lambda i, _nd=a.ndim: (0,) * _nd)
                for a in args]
    out = pl.pallas_call(
        _head_body,
        out_shape=jax.ShapeDtypeStruct((B, 128), jnp.float32),
        grid=(1,),
        in_specs=in_specs,
        out_specs=pl.BlockSpec((B, 128), lambda i: (0, 0)),
        compiler_params=pltpu.CompilerParams(
            vmem_limit_bytes=64 * 1024 * 1024),
    )(*args)
    return out[:, :50]


def _bottleneck0(x, p, stride):
    idn = _conv1x1(x, p["wd"], p["sd"], p["td"], stride=stride, relu=False)
    h = _conv1x1(x, p["w1"], p["s1"], p["t1"], relu=True)
    h = _conv_knxn(h, p["w2"], p["s2"], p["t2"],
                   stride=stride, padding=1, relu=True)
    return _conv1x1(h, p["w3"], p["s3"], p["t3"], relu=True, residual=idn)


def kernel(conv1_w, bn1_s, bn1_sh,
           l1_b0_w1, l1_b0_s1, l1_b0_sh1, l1_b0_w2, l1_b0_s2, l1_b0_sh2,
           l1_b0_w3, l1_b0_s3, l1_b0_sh3, l1_b0_wd, l1_b0_sd, l1_b0_shd,
           l1_rest_w1, l1_rest_s1, l1_rest_sh1, l1_rest_w2, l1_rest_s2,
           l1_rest_sh2, l1_rest_w3, l1_rest_s3, l1_rest_sh3,
           l2_b0_w1, l2_b0_s1, l2_b0_sh1, l2_b0_w2, l2_b0_s2, l2_b0_sh2,
           l2_b0_w3, l2_b0_s3, l2_b0_sh3, l2_b0_wd, l2_b0_sd, l2_b0_shd,
           l2_rest_w1, l2_rest_s1, l2_rest_sh1, l2_rest_w2, l2_rest_s2,
           l2_rest_sh2, l2_rest_w3, l2_rest_s3, l2_rest_sh3,
           l3_b0_w1, l3_b0_s1, l3_b0_sh1, l3_b0_w2, l3_b0_s2, l3_b0_sh2,
           l3_b0_w3, l3_b0_s3, l3_b0_sh3, l3_b0_wd, l3_b0_sd, l3_b0_shd,
           l3_rest_w1, l3_rest_s1, l3_rest_sh1, l3_rest_w2, l3_rest_s2,
           l3_rest_sh2, l3_rest_w3, l3_rest_s3, l3_rest_sh3,
           wr, br, wu, bu, wfr, wfu, bf,
           x_nchw, y):
    x = _stem_conv(x_nchw, conv1_w, bn1_s, bn1_sh)

    layers = (
        (1, {"w1": l1_b0_w1, "s1": l1_b0_s1, "t1": l1_b0_sh1,
             "w2": l1_b0_w2, "s2": l1_b0_s2, "t2": l1_b0_sh2,
             "w3": l1_b0_w3, "s3": l1_b0_s3, "t3": l1_b0_sh3,
             "wd": l1_b0_wd, "sd": l1_b0_sd, "td": l1_b0_shd},
         (l1_rest_w1, l1_rest_s1, l1_rest_sh1, l1_rest_w2, l1_rest_s2,
          l1_rest_sh2, l1_rest_w3, l1_rest_s3, l1_rest_sh3)),
        (2, {"w1": l2_b0_w1, "s1": l2_b0_s1, "t1": l2_b0_sh1,
             "w2": l2_b0_w2, "s2": l2_b0_s2, "t2": l2_b0_sh2,
             "w3": l2_b0_w3, "s3": l2_b0_s3, "t3": l2_b0_sh3,
             "wd": l2_b0_wd, "sd": l2_b0_sd, "td": l2_b0_shd},
         (l2_rest_w1, l2_rest_s1, l2_rest_sh1, l2_rest_w2, l2_rest_s2,
          l2_rest_sh2, l2_rest_w3, l2_rest_s3, l2_rest_sh3)),
        (2, {"w1": l3_b0_w1, "s1": l3_b0_s1, "t1": l3_b0_sh1,
             "w2": l3_b0_w2, "s2": l3_b0_s2, "t2": l3_b0_sh2,
             "w3": l3_b0_w3, "s3": l3_b0_s3, "t3": l3_b0_sh3,
             "wd": l3_b0_wd, "sd": l3_b0_sd, "td": l3_b0_shd},
         (l3_rest_w1, l3_rest_s1, l3_rest_sh1, l3_rest_w2, l3_rest_s2,
          l3_rest_sh2, l3_rest_w3, l3_rest_s3, l3_rest_sh3)),
    )

    for stride0, b0, rest in layers:
        x = _b0_fused(x, b0, stride0)
        x = _bottleneck_stack(x, *rest)

    B = x.shape[0]
    feat = x.reshape(B, x.shape[1] * x.shape[2], x.shape[3])
    y2 = y.reshape(y.shape[0], -1).astype(BF16)
    y2 = jnp.pad(y2, ((0, 0), (0, 256 - y2.shape[1])))
    return _head(feat, y2, wr, br, wu, bu, wfr, wfu, bf)


# speedup vs baseline: 1.0505x; 1.0505x over previous
import functools
import math

import numpy as np

import jax
import jax.numpy as jnp
from jax.experimental import pallas as pl
from jax.experimental.pallas import tpu as pltpu

BF16 = jnp.bfloat16


def _ru(x, m):
    return ((x + m - 1) // m) * m


def _pad2d(a, rows, cols):
    if a.shape[0] == rows and a.shape[1] == cols:
        return a
    return jnp.pad(a, ((0, rows - a.shape[0]), (0, cols - a.shape[1])))


def _mm_body(a_ref, b_ref, s_ref, t_ref, *rest, relu, has_res):
    if has_res:
        r_ref, o_ref = rest
    else:
        (o_ref,) = rest
    acc = jnp.dot(a_ref[...], b_ref[...], preferred_element_type=jnp.float32)
    acc = acc * s_ref[...] + t_ref[...]
    if has_res:
        acc = acc + r_ref[...].astype(jnp.float32)
    if relu:
        acc = jnp.maximum(acc, 0.0)
    o_ref[...] = acc.astype(o_ref.dtype)


def _mm_bn(a, w, s, t, *, relu, residual=None, out_dtype=BF16):
    M, K = a.shape
    N = w.shape[1]
    a = a.astype(BF16)
    w = w.astype(BF16)

    bm = min(512, _ru(M, 8))
    bn = min(256, _ru(N, 128))
    Kp = _ru(K, 128)
    if M // bm < 2 and bm > 8:
        bm = max(8, _ru(bm // 2, 8))
    Mp, Np = _ru(M, bm), _ru(N, bn)

    args = [_pad2d(a, Mp, Kp), _pad2d(w, Kp, Np),
            _pad2d(s, 1, Np), _pad2d(t, 1, Np)]
    has_res = residual is not None
    if has_res:
        args.append(_pad2d(residual.astype(BF16), Mp, Np))

    in_specs = [pl.BlockSpec((bm, Kp), lambda i, j: (i, 0)),
                pl.BlockSpec((Kp, bn), lambda i, j: (0, j)),
                pl.BlockSpec((1, bn), lambda i, j: (0, j)),
                pl.BlockSpec((1, bn), lambda i, j: (0, j))]
    if has_res:
        in_specs.append(pl.BlockSpec((bm, bn), lambda i, j: (i, j)))

    out = pl.pallas_call(
        functools.partial(_mm_body, relu=relu, has_res=has_res),
        out_shape=jax.ShapeDtypeStruct((Mp, Np), out_dtype),
        grid=(Mp // bm, Np // bn),
        in_specs=in_specs,
        out_specs=pl.BlockSpec((bm, bn), lambda i, j: (i, j)),
        compiler_params=pltpu.CompilerParams(
            dimension_semantics=("parallel", "parallel"),
            vmem_limit_bytes=64 * 1024 * 1024),
    )(*args)
    return out[:M, :N]


def _conv1x1(x, w, s, t, *, stride=1, relu, residual=None):
    if stride > 1:
        x = x[:, ::stride, ::stride, :]
    B, H, W, C = x.shape
    N = w.shape[-1]
    res = None if residual is None else residual.reshape(B * H * W, N)
    out = _mm_bn(x.reshape(B * H * W, C), w, s, t, relu=relu, residual=res)
    return out.reshape(B, H, W, N)


def _conv_knxn(x, w, s, t, *, stride, padding, relu):
    B, H, W, C = x.shape
    KH, KW, _, N = w.shape
    xp = jnp.pad(x, ((0, 0), (padding, padding), (padding, padding), (0, 0)))
    OH = (H + 2 * padding - KH) // stride + 1
    OW = (W + 2 * padding - KW) // stride + 1
    taps = [xp[:, kh:kh + stride * (OH - 1) + 1:stride,
               kw:kw + stride * (OW - 1) + 1:stride, :]
            for kh in range(KH) for kw in range(KW)]
    patches = jnp.concatenate(taps, axis=-1).reshape(B * OH * OW, KH * KW * C)
    out = _mm_bn(patches, w.reshape(KH * KW * C, N), s, t, relu=relu)
    return out.reshape(B, OH, OW, N)


def _maxpool_3x3_s2(x):
    init = jnp.array(-jnp.inf, dtype=x.dtype)
    return jax.lax.reduce_window(
        x, init, jax.lax.max,
        window_dimensions=(1, 3, 3, 1), window_strides=(1, 2, 2, 1),
        padding=((0, 0), (1, 1), (1, 1), (0, 0)))


def _stack_body(x0_ref, m_ref, w1_ref, s1_ref, t1_ref,
                w2_ref, s2_ref, t2_ref, w3_ref, s3_ref, t3_ref,
                o_ref, h_ref, *, g, wrow, rows):
    blk = pl.program_id(1)

    @pl.when(blk == 0)
    def _():
        o_ref[...] = x0_ref[...]

    act = o_ref[0]

    h1 = jnp.dot(act, w1_ref[0], preferred_element_type=jnp.float32)
    h1 = jnp.maximum(h1 * s1_ref[0] + t1_ref[0], 0.0) * m_ref[...]
    h_ref[...] = h1.astype(BF16)

    shifts = [h_ref[g + dh * wrow + dw:g + dh * wrow + dw + rows, :]
              for dh in (-1, 0, 1) for dw in (-1, 0, 1)]
    wide = jnp.concatenate(shifts, axis=1)
    h2 = jnp.dot(wide, w2_ref[0], preferred_element_type=jnp.float32)
    h2 = jnp.maximum(h2 * s2_ref[0] + t2_ref[0], 0.0).astype(BF16)

    o3 = jnp.dot(h2, w3_ref[0], preferred_element_type=jnp.float32)
    o3 = o3 * s3_ref[0] + t3_ref[0] + act[g:g + rows, :].astype(jnp.float32)
    o_ref[0, g:g + rows, :] = jnp.maximum(o3, 0.0).astype(BF16)


def _fold_groups(x, G, n_pad):
    B, H, W, C = x.shape
    n = (H + 2) * (W + 2)
    xp = jnp.pad(x, ((0, 0), (1, 1), (1, 1), (0, 0))).reshape(B, n, C)
    xp = jnp.pad(xp, ((0, 0), (G, n_pad - n), (0, 0)))
    xp = xp.reshape(2, (B // 2) * (G + n_pad), C)
    return jnp.pad(xp, ((0, 0), (0, G), (0, 0)))


def _unfold_groups(o, B, H, W, G, n_pad):
    Hp, Wp = H + 2, W + 2
    seg = G + n_pad
    body = o[:, :(B // 2) * seg, :].reshape(B, seg, -1)[:, G:G + Hp * Wp, :]
    return body.reshape(B, Hp, Wp, -1)[:, 1:H + 1, 1:W + 1, :]


def _row_mask(H, W, G, n_pad, per_group):
    Hp, Wp = H + 2, W + 2
    m = np.zeros((Hp, Wp), np.float32)
    m[1:H + 1, 1:W + 1] = 1.0
    seg = np.concatenate([np.zeros(G, np.float32), m.reshape(-1),
                          np.zeros(n_pad - Hp * Wp, np.float32)])
    full = np.concatenate([np.tile(seg, per_group), np.zeros(G, np.float32)])
    return jnp.asarray(full.reshape(-1, 1))


def _bottleneck_stack(x, w1, s1, t1, w2, s2, t2, w3, s3, t3):
    B, H, W, C = x.shape
    NB, _, Pp = w1.shape
    Wp = W + 2
    n = (H + 2) * Wp
    n_pad = _ru(n, 8)
    G = _ru(Wp + 1, 8)
    Rg = (B // 2) * (G + n_pad) + G
    rows = Rg - 2 * G

    x0 = _fold_groups(x.astype(BF16), G, n_pad)
    mask = _row_mask(H, W, G, n_pad, B // 2)
    w2k = w2.reshape(NB, 9 * Pp, Pp)

    out = pl.pallas_call(
        functools.partial(_stack_body, g=G, wrow=Wp, rows=rows),
        out_shape=jax.ShapeDtypeStruct((2, Rg, C), BF16),
        grid_spec=pltpu.PrefetchScalarGridSpec(
            num_scalar_prefetch=0,
            grid=(2, NB),
            in_specs=[
                pl.BlockSpec((1, Rg, C), lambda g, b: (g, 0, 0)),
                pl.BlockSpec((Rg, 1), lambda g, b: (0, 0)),
                pl.BlockSpec((1, C, Pp), lambda g, b: (b, 0, 0)),
                pl.BlockSpec((1, 1, Pp), lambda g, b: (b, 0, 0)),
                pl.BlockSpec((1, 1, Pp), lambda g, b: (b, 0, 0)),
                pl.BlockSpec((1, 9 * Pp, Pp), lambda g, b: (b, 0, 0)),
                pl.BlockSpec((1, 1, Pp), lambda g, b: (b, 0, 0)),
                pl.BlockSpec((1, 1, Pp), lambda g, b: (b, 0, 0)),
                pl.BlockSpec((1, Pp, C), lambda g, b: (b, 0, 0)),
                pl.BlockSpec((1, 1, C), lambda g, b: (b, 0, 0)),
                pl.BlockSpec((1, 1, C), lambda g, b: (b, 0, 0)),
            ],
            out_specs=pl.BlockSpec((1, Rg, C), lambda g, b: (g, 0, 0)),
            scratch_shapes=[pltpu.VMEM((Rg, Pp), BF16)]),
        compiler_params=pltpu.CompilerParams(
            dimension_semantics=("parallel", "arbitrary"),
            vmem_limit_bytes=64 * 1024 * 1024),
    )(x0, mask, w1, s1, t1, w2k, s2, t2, w3, s3, t3)

    return _unfold_groups(out, B, H, W, G, n_pad)


def _b0_body(x_ref, m_ref, w1_ref, s1_ref, t1_ref, w2_ref, s2_ref, t2_ref,
             w3_ref, s3_ref, t3_ref, wd_ref, sd_ref, td_ref, o_ref,
             *, g, bm, nph, cin, pp, taps):
    base = pl.program_id(1) * bm
    xw = x_ref[0, pl.ds(base, bm + 2 * g), :]
    mw = m_ref[pl.ds(base, bm + 2 * g), :]

    hs = []
    for p in range(nph):
        hp = jnp.dot(xw[:, p * cin:(p + 1) * cin], w1_ref[...],
                     preferred_element_type=jnp.float32)
        hs.append((jnp.maximum(hp * s1_ref[...] + t1_ref[...], 0.0)
                   * mw).astype(BF16))

    wide = jnp.concatenate(
        [hs[b][g + s:g + s + bm, :] for s, b in taps], axis=1)
    h2 = jnp.dot(wide, w2_ref[...], preferred_element_type=jnp.float32)
    h2 = jnp.maximum(h2 * s2_ref[...] + t2_ref[...], 0.0).astype(BF16)

    o3 = jnp.dot(h2, w3_ref[...], preferred_element_type=jnp.float32)
    o3 = o3 * s3_ref[...] + t3_ref[...]
    idn = jnp.dot(xw[g:g + bm, :cin], wd_ref[...],
                  preferred_element_type=jnp.float32)
    o3 = o3 + idn * sd_ref[...] + td_ref[...]
    o_ref[0] = jnp.maximum(o3, 0.0).astype(BF16)


def _pad_last(a, n):
    if a.shape[-1] == n:
        return a
    return jnp.pad(a, [(0, 0)] * (a.ndim - 1) + [(0, n - a.shape[-1])])


def _b0_fused(x, p, stride):
    B, H, W, C = x.shape
    if stride == 2:
        H, W = H // 2, W // 2
        x = x.reshape(B, H, 2, W, 2, C).transpose(0, 1, 3, 2, 4, 5)
        x = x.reshape(B, H, W, 4 * C)
        nph = 4
    else:
        nph = 1
    P0 = p["w1"].shape[1]
    C2 = p["w3"].shape[1]
    P = max(P0, 128)
    Cp = max(C, 128)
    x = _pad_last(x, nph * Cp) if C < 128 else x

    Wp = W + 2
    n = (H + 2) * Wp
    n_pad = _ru(n, 8)
    G = _ru(Wp + 1, 8)
    Rg = (B // 2) * (G + n_pad) + G
    T = -(-Rg // 1536)
    bm = _ru(-(-Rg // T), 8)
    Rgp = bm * T

    x0 = _fold_groups(x.astype(BF16), G, n_pad)
    x0 = jnp.pad(x0, ((0, 0), (0, Rgp + 2 * G - Rg), (0, 0)))
    mask = _row_mask(H, W, G, n_pad, B // 2)
    mask = jnp.pad(mask, ((0, Rgp + 2 * G - Rg), (0, 0)))

    w1 = _pad2d(p["w1"].astype(BF16), Cp, P)
    s1, t1 = _pad2d(p["s1"], 1, P), _pad2d(p["t1"], 1, P)
    w2 = p["w2"].reshape(3, 3, P0, P0).reshape(9, P0, P0)
    w2 = jnp.pad(w2, ((0, 0), (0, P - P0), (0, P - P0)))
    w2 = w2.reshape(9 * P, P).astype(BF16)
    s2, t2 = _pad2d(p["s2"], 1, P), _pad2d(p["t2"], 1, P)
    w3 = _pad2d(p["w3"].astype(BF16), P, C2)
    wd = _pad2d(p["wd"].astype(BF16), Cp, C2)

    if nph == 1:
        taps = [(dh * Wp + dw, 0) for dh in (-1, 0, 1) for dw in (-1, 0, 1)]
    else:
        def phm(k):
            return (1, -1) if k == -1 else (0, 0) if k == 0 else (1, 0)
        taps = []
        for kh in (-1, 0, 1):
            ph, di = phm(kh)
            for kw in (-1, 0, 1):
                pw, dj = phm(kw)
                taps.append((di * Wp + dj, ph * 2 + pw))

    Cx = x0.shape[-1]
    out = pl.pallas_call(
        functools.partial(_b0_body, g=G, bm=bm, nph=nph, cin=Cp, pp=P,
                          taps=taps),
        out_shape=jax.ShapeDtypeStruct((2, Rgp, C2), BF16),
        grid=(2, T),
        in_specs=[pl.BlockSpec((1, Rgp + 2 * G, Cx), lambda g, i: (g, 0, 0)),
                  pl.BlockSpec((Rgp + 2 * G, 1), lambda g, i: (0, 0)),
                  pl.BlockSpec((Cp, P), lambda g, i: (0, 0)),
                  pl.BlockSpec((1, P), lambda g, i: (0, 0)),
                  pl.BlockSpec((1, P), lambda g, i: (0, 0)),
                  pl.BlockSpec((9 * P, P), lambda g, i: (0, 0)),
                  pl.BlockSpec((1, P), lambda g, i: (0, 0)),
                  pl.BlockSpec((1, P), lambda g, i: (0, 0)),
                  pl.BlockSpec((P, C2), lambda g, i: (0, 0)),
                  pl.BlockSpec((1, C2), lambda g, i: (0, 0)),
                  pl.BlockSpec((1, C2), lambda g, i: (0, 0)),
                  pl.BlockSpec((Cp, C2), lambda g, i: (0, 0)),
                  pl.BlockSpec((1, C2), lambda g, i: (0, 0)),
                  pl.BlockSpec((1, C2), lambda g, i: (0, 0))],
        out_specs=pl.BlockSpec((1, bm, C2), lambda g, i: (g, i, 0)),
        compiler_params=pltpu.CompilerParams(
            dimension_semantics=("parallel", "parallel"),
            vmem_limit_bytes=64 * 1024 * 1024),
    )(x0, mask, w1, s1, t1, w2, s2, t2, w3, p["s3"], p["t3"],
      wd, p["sd"], p["td"])

    seg = G + n_pad
    body = out.reshape(2, Rgp, C2)[:, :(B // 2) * seg, :].reshape(B, seg, C2)
    body = body[:, :n, :].reshape(B, H + 2, Wp, C2)
    return body[:, 1:H + 1, 1:W + 1, :]


def _stem_body(x_ref, w_ref, s_ref, t_ref, o_ref, *, wrow, bm):
    base = pl.program_id(1) * bm
    win = x_ref[0, pl.ds(base, bm + 256), :]
    shifts = [dh * wrow + dw for dh in range(4) for dw in range(4)]
    wide = jnp.concatenate([win[s:s + bm, :] for s in shifts], axis=1)
    h = jnp.dot(wide, w_ref[...], preferred_element_type=jnp.float32)
    o_ref[0] = jnp.maximum(h * s_ref[...] + t_ref[...], 0.0).astype(BF16)


def _stem_conv(x_nchw, w, s, t):
    B = x_nchw.shape[0]
    x = x_nchw.astype(BF16)
    x = jnp.pad(x, ((0, 0), (0, 1), (3, 3), (3, 3)))
    x = x.reshape(B, 4, 67, 2, 67, 2).transpose(0, 2, 4, 3, 5, 1)
    x = x.reshape(B, 67, 67, 16)
    n, n_pad, tail, bm = 67 * 67, 4608, 256, 2304
    rows = (B // 2) * n_pad
    x = jnp.pad(x.reshape(B, n, 16), ((0, 0), (0, n_pad - n), (0, 0)))
    x = x.reshape(2, rows, 16)
    x = jnp.pad(x, ((0, 0), (0, tail), (0, 0)))

    wp = jnp.pad(w, ((0, 1), (0, 1), (0, 1), (0, 0)))
    wk = wp.reshape(4, 2, 4, 2, 4, 64).transpose(0, 2, 1, 3, 4, 5)
    wk = wk.reshape(256, 64).astype(BF16)
    wk = jnp.pad(wk, ((0, 0), (0, 64)))
    sp = _pad2d(s, 1, 128)
    tp = _pad2d(t, 1, 128)

    out = pl.pallas_call(
        functools.partial(_stem_body, wrow=67, bm=bm),
        out_shape=jax.ShapeDtypeStruct((2, rows, 128), BF16),
        grid=(2, rows // bm),
        in_specs=[pl.BlockSpec((1, rows + tail, 16), lambda g, i: (g, 0, 0)),
                  pl.BlockSpec((256, 128), lambda g, i: (0, 0)),
                  pl.BlockSpec((1, 128), lambda g, i: (0, 0)),
                  pl.BlockSpec((1, 128), lambda g, i: (0, 0))],
        out_specs=pl.BlockSpec((1, bm, 128), lambda g, i: (g, i, 0)),
        compiler_params=pltpu.CompilerParams(
            dimension_semantics=("parallel", "parallel"),
            vmem_limit_bytes=64 * 1024 * 1024),
    )(x, wk, sp, tp)

    out = out.reshape(B, n_pad, 128)[:, :n, :].reshape(B, 67, 67, 128)
    return out[:, :64, :64, :64]


def _head_body(f_ref, y_ref, wr_ref, br_ref, wu_ref, bu_ref,
               wfr_ref, wfu_ref, bf_ref, o_ref):
    pooled = jnp.mean(f_ref[...].astype(jnp.float32), axis=1)
    r = jnp.dot(pooled.astype(BF16), wr_ref[...],
                preferred_element_type=jnp.float32) + br_ref[...]
    u = jnp.dot(y_ref[...], wu_ref[...],
                preferred_element_type=jnp.float32) + bu_ref[...]
    z = (jnp.dot(r.astype(BF16), wfr_ref[...],
                 preferred_element_type=jnp.float32)
         + jnp.dot(u.astype(BF16), wfu_ref[...],
                   preferred_element_type=jnp.float32)
         + bf_ref[...])
    o_ref[...] = z


def _head(feat, y2, wr, br, wu, bu, wfr, wfu, bf):
    B = feat.shape[0]
    args = (feat, y2, wr, br, wu, bu, wfr, wfu, bf)
    in_specs = [pl.BlockSpec(a.shape, lambda i, _nd=a.ndim: (0,) * _nd)
                for a in args]
    out = pl.pallas_call(
        _head_body,
        out_shape=jax.ShapeDtypeStruct((B, 128), jnp.float32),
        grid=(1,),
        in_specs=in_specs,
        out_specs=pl.BlockSpec((B, 128), lambda i: (0, 0)),
        compiler_params=pltpu.CompilerParams(
            vmem_limit_bytes=64 * 1024 * 1024),
    )(*args)
    return out[:, :50]


def _bottleneck0(x, p, stride):
    idn = _conv1x1(x, p["wd"], p["sd"], p["td"], stride=stride, relu=False)
    h = _conv1x1(x, p["w1"], p["s1"], p["t1"], relu=True)
    h = _conv_knxn(h, p["w2"], p["s2"], p["t2"],
                   stride=stride, padding=1, relu=True)
    return _conv1x1(h, p["w3"], p["s3"], p["t3"], relu=True, residual=idn)


def kernel(conv1_w, bn1_s, bn1_sh,
           l1_b0_w1, l1_b0_s1, l1_b0_sh1, l1_b0_w2, l1_b0_s2, l1_b0_sh2,
           l1_b0_w3, l1_b0_s3, l1_b0_sh3, l1_b0_wd, l1_b0_sd, l1_b0_shd,
           l1_rest_w1, l1_rest_s1, l1_rest_sh1, l1_rest_w2, l1_rest_s2,
           l1_rest_sh2, l1_rest_w3, l1_rest_s3, l1_rest_sh3,
           l2_b0_w1, l2_b0_s1, l2_b0_sh1, l2_b0_w2, l2_b0_s2, l2_b0_sh2,
           l2_b0_w3, l2_b0_s3, l2_b0_sh3, l2_b0_wd, l2_b0_sd, l2_b0_shd,
           l2_rest_w1, l2_rest_s1, l2_rest_sh1, l2_rest_w2, l2_rest_s2,
           l2_rest_sh2, l2_rest_w3, l2_rest_s3, l2_rest_sh3,
           l3_b0_w1, l3_b0_s1, l3_b0_sh1, l3_b0_w2, l3_b0_s2, l3_b0_sh2,
           l3_b0_w3, l3_b0_s3, l3_b0_sh3, l3_b0_wd, l3_b0_sd, l3_b0_shd,
           l3_rest_w1, l3_rest_s1, l3_rest_sh1, l3_rest_w2, l3_rest_s2,
           l3_rest_sh2, l3_rest_w3, l3_rest_s3, l3_rest_sh3,
           wr, br, wu, bu, wfr, wfu, bf,
           x_nchw, y):
    x = _stem_conv(x_nchw, conv1_w, bn1_s, bn1_sh)
    x = _maxpool_3x3_s2(x)

    layers = (
        (1, {"w1": l1_b0_w1, "s1": l1_b0_s1, "t1": l1_b0_sh1,
             "w2": l1_b0_w2, "s2": l1_b0_s2, "t2": l1_b0_sh2,
             "w3": l1_b0_w3, "s3": l1_b0_s3, "t3": l1_b0_sh3,
             "wd": l1_b0_wd, "sd": l1_b0_sd, "td": l1_b0_shd},
         (l1_rest_w1, l1_rest_s1, l1_rest_sh1, l1_rest_w2, l1_rest_s2,
          l1_rest_sh2, l1_rest_w3, l1_rest_s3, l1_rest_sh3)),
        (2, {"w1": l2_b0_w1, "s1": l2_b0_s1, "t1": l2_b0_sh1,
             "w2": l2_b0_w2, "s2": l2_b0_s2, "t2": l2_b0_sh2,
             "w3": l2_b0_w3, "s3": l2_b0_s3, "t3": l2_b0_sh3,
             "wd": l2_b0_wd, "sd": l2_b0_sd, "td": l2_b0_shd},
         (l2_rest_w1, l2_rest_s1, l2_rest_sh1, l2_rest_w2, l2_rest_s2,
          l2_rest_sh2, l2_rest_w3, l2_rest_s3, l2_rest_sh3)),
        (2, {"w1": l3_b0_w1, "s1": l3_b0_s1, "t1": l3_b0_sh1,
             "w2": l3_b0_w2, "s2": l3_b0_s2, "t2": l3_b0_sh2,
             "w3": l3_b0_w3, "s3": l3_b0_s3, "t3": l3_b0_sh3,
             "wd": l3_b0_wd, "sd": l3_b0_sd, "td": l3_b0_shd},
         (l3_rest_w1, l3_rest_s1, l3_rest_sh1, l3_rest_w2, l3_rest_s2,
          l3_rest_sh2, l3_rest_w3, l3_rest_s3, l3_rest_sh3)),
    )

    for stride0, b0, rest in layers:
        x = _b0_fused(x, b0, stride0)
        x = _bottleneck_stack(x, *rest)

    B = x.shape[0]
    feat = x.reshape(B, x.shape[1] * x.shape[2], x.shape[3])
    y2 = y.reshape(y.shape[0], -1).astype(BF16)
    y2 = jnp.pad(y2, ((0, 0), (0, 256 - y2.shape[1])))
    return _head(feat, y2, wr, br, wu, bu, wfr, wfu, bf)


# speedup vs baseline: 1.0596x; 1.0087x over previous
import functools
import math

import numpy as np

import jax
import jax.numpy as jnp
from jax.experimental import pallas as pl
from jax.experimental.pallas import tpu as pltpu

BF16 = jnp.bfloat16


def _ru(x, m):
    return ((x + m - 1) // m) * m


def _pad2d(a, rows, cols):
    if a.shape[0] == rows and a.shape[1] == cols:
        return a
    return jnp.pad(a, ((0, rows - a.shape[0]), (0, cols - a.shape[1])))


def _mm_body(a_ref, b_ref, s_ref, t_ref, *rest, relu, has_res):
    if has_res:
        r_ref, o_ref = rest
    else:
        (o_ref,) = rest
    acc = jnp.dot(a_ref[...], b_ref[...], preferred_element_type=jnp.float32)
    acc = acc * s_ref[...] + t_ref[...]
    if has_res:
        acc = acc + r_ref[...].astype(jnp.float32)
    if relu:
        acc = jnp.maximum(acc, 0.0)
    o_ref[...] = acc.astype(o_ref.dtype)


def _mm_bn(a, w, s, t, *, relu, residual=None, out_dtype=BF16):
    M, K = a.shape
    N = w.shape[1]
    a = a.astype(BF16)
    w = w.astype(BF16)

    bm = min(512, _ru(M, 8))
    bn = min(256, _ru(N, 128))
    Kp = _ru(K, 128)
    if M // bm < 2 and bm > 8:
        bm = max(8, _ru(bm // 2, 8))
    Mp, Np = _ru(M, bm), _ru(N, bn)

    args = [_pad2d(a, Mp, Kp), _pad2d(w, Kp, Np),
            _pad2d(s, 1, Np), _pad2d(t, 1, Np)]
    has_res = residual is not None
    if has_res:
        args.append(_pad2d(residual.astype(BF16), Mp, Np))

    in_specs = [pl.BlockSpec((bm, Kp), lambda i, j: (i, 0)),
                pl.BlockSpec((Kp, bn), lambda i, j: (0, j)),
                pl.BlockSpec((1, bn), lambda i, j: (0, j)),
                pl.BlockSpec((1, bn), lambda i, j: (0, j))]
    if has_res:
        in_specs.append(pl.BlockSpec((bm, bn), lambda i, j: (i, j)))

    out = pl.pallas_call(
        functools.partial(_mm_body, relu=relu, has_res=has_res),
        out_shape=jax.ShapeDtypeStruct((Mp, Np), out_dtype),
        grid=(Mp // bm, Np // bn),
        in_specs=in_specs,
        out_specs=pl.BlockSpec((bm, bn), lambda i, j: (i, j)),
        compiler_params=pltpu.CompilerParams(
            dimension_semantics=("parallel", "parallel"),
            vmem_limit_bytes=64 * 1024 * 1024),
    )(*args)
    return out[:M, :N]


def _conv1x1(x, w, s, t, *, stride=1, relu, residual=None):
    if stride > 1:
        x = x[:, ::stride, ::stride, :]
    B, H, W, C = x.shape
    N = w.shape[-1]
    res = None if residual is None else residual.reshape(B * H * W, N)
    out = _mm_bn(x.reshape(B * H * W, C), w, s, t, relu=relu, residual=res)
    return out.reshape(B, H, W, N)


def _conv_knxn(x, w, s, t, *, stride, padding, relu):
    B, H, W, C = x.shape
    KH, KW, _, N = w.shape
    xp = jnp.pad(x, ((0, 0), (padding, padding), (padding, padding), (0, 0)))
    OH = (H + 2 * padding - KH) // stride + 1
    OW = (W + 2 * padding - KW) // stride + 1
    taps = [xp[:, kh:kh + stride * (OH - 1) + 1:stride,
               kw:kw + stride * (OW - 1) + 1:stride, :]
            for kh in range(KH) for kw in range(KW)]
    patches = jnp.concatenate(taps, axis=-1).reshape(B * OH * OW, KH * KW * C)
    out = _mm_bn(patches, w.reshape(KH * KW * C, N), s, t, relu=relu)
    return out.reshape(B, OH, OW, N)


def _maxpool_3x3_s2(x):
    init = jnp.array(-jnp.inf, dtype=x.dtype)
    return jax.lax.reduce_window(
        x, init, jax.lax.max,
        window_dimensions=(1, 3, 3, 1), window_strides=(1, 2, 2, 1),
        padding=((0, 0), (1, 1), (1, 1), (0, 0)))


def _stack_body(x0_ref, m_ref, w1_ref, s1_ref, t1_ref,
                w2_ref, s2_ref, t2_ref, w3_ref, s3_ref, t3_ref,
                o_ref, h_ref, *, g, wrow, rows):
    blk = pl.program_id(1)

    @pl.when(blk == 0)
    def _():
        o_ref[...] = x0_ref[...]

    act = o_ref[0]

    h1 = jnp.dot(act, w1_ref[0], preferred_element_type=jnp.float32)
    h1 = jnp.maximum(h1 * s1_ref[0] + t1_ref[0], 0.0) * m_ref[...]
    h_ref[...] = h1.astype(BF16)

    shifts = [h_ref[g + dh * wrow + dw:g + dh * wrow + dw + rows, :]
              for dh in (-1, 0, 1) for dw in (-1, 0, 1)]
    wide = jnp.concatenate(shifts, axis=1)
    h2 = jnp.dot(wide, w2_ref[0], preferred_element_type=jnp.float32)
    h2 = jnp.maximum(h2 * s2_ref[0] + t2_ref[0], 0.0).astype(BF16)

    o3 = jnp.dot(h2, w3_ref[0], preferred_element_type=jnp.float32)
    o3 = o3 * s3_ref[0] + t3_ref[0] + act[g:g + rows, :].astype(jnp.float32)
    o_ref[0, g:g + rows, :] = jnp.maximum(o3, 0.0).astype(BF16)


def _fold_groups(x, G, n_pad):
    B, H, W, C = x.shape
    n = (H + 2) * (W + 2)
    xp = jnp.pad(x, ((0, 0), (1, 1), (1, 1), (0, 0))).reshape(B, n, C)
    xp = jnp.pad(xp, ((0, 0), (G, n_pad - n), (0, 0)))
    xp = xp.reshape(2, (B // 2) * (G + n_pad), C)
    return jnp.pad(xp, ((0, 0), (0, G), (0, 0)))


def _unfold_groups(o, B, H, W, G, n_pad):
    Hp, Wp = H + 2, W + 2
    seg = G + n_pad
    body = o[:, :(B // 2) * seg, :].reshape(B, seg, -1)[:, G:G + Hp * Wp, :]
    return body.reshape(B, Hp, Wp, -1)[:, 1:H + 1, 1:W + 1, :]


def _row_mask(H, W, G, n_pad, per_group):
    Hp, Wp = H + 2, W + 2
    m = np.zeros((Hp, Wp), np.float32)
    m[1:H + 1, 1:W + 1] = 1.0
    seg = np.concatenate([np.zeros(G, np.float32), m.reshape(-1),
                          np.zeros(n_pad - Hp * Wp, np.float32)])
    full = np.concatenate([np.tile(seg, per_group), np.zeros(G, np.float32)])
    return jnp.asarray(full.reshape(-1, 1))


def _bottleneck_stack(x, w1, s1, t1, w2, s2, t2, w3, s3, t3,
                      keep_folded=False):
    B, H, W, C = x.shape
    NB, _, Pp = w1.shape
    Wp = W + 2
    n = (H + 2) * Wp
    n_pad = _ru(n, 8)
    G = _ru(Wp + 1, 8)
    Rg = (B // 2) * (G + n_pad) + G
    rows = Rg - 2 * G

    x0 = _fold_groups(x.astype(BF16), G, n_pad)
    mask = _row_mask(H, W, G, n_pad, B // 2)
    w2k = w2.reshape(NB, 9 * Pp, Pp)

    out = pl.pallas_call(
        functools.partial(_stack_body, g=G, wrow=Wp, rows=rows),
        out_shape=jax.ShapeDtypeStruct((2, Rg, C), BF16),
        grid_spec=pltpu.PrefetchScalarGridSpec(
            num_scalar_prefetch=0,
            grid=(2, NB),
            in_specs=[
                pl.BlockSpec((1, Rg, C), lambda g, b: (g, 0, 0)),
                pl.BlockSpec((Rg, 1), lambda g, b: (0, 0)),
                pl.BlockSpec((1, C, Pp), lambda g, b: (b, 0, 0)),
                pl.BlockSpec((1, 1, Pp), lambda g, b: (b, 0, 0)),
                pl.BlockSpec((1, 1, Pp), lambda g, b: (b, 0, 0)),
                pl.BlockSpec((1, 9 * Pp, Pp), lambda g, b: (b, 0, 0)),
                pl.BlockSpec((1, 1, Pp), lambda g, b: (b, 0, 0)),
                pl.BlockSpec((1, 1, Pp), lambda g, b: (b, 0, 0)),
                pl.BlockSpec((1, Pp, C), lambda g, b: (b, 0, 0)),
                pl.BlockSpec((1, 1, C), lambda g, b: (b, 0, 0)),
                pl.BlockSpec((1, 1, C), lambda g, b: (b, 0, 0)),
            ],
            out_specs=pl.BlockSpec((1, Rg, C), lambda g, b: (g, 0, 0)),
            scratch_shapes=[pltpu.VMEM((Rg, Pp), BF16)]),
        compiler_params=pltpu.CompilerParams(
            dimension_semantics=("parallel", "arbitrary"),
            vmem_limit_bytes=64 * 1024 * 1024),
    )(x0, mask, w1, s1, t1, w2k, s2, t2, w3, s3, t3)

    if keep_folded:
        return out, mask, G + n_pad, H * W
    return _unfold_groups(out, B, H, W, G, n_pad)


def _b0_body(x_ref, m_ref, w1_ref, s1_ref, t1_ref, w2_ref, s2_ref, t2_ref,
             w3_ref, s3_ref, t3_ref, wd_ref, sd_ref, td_ref, o_ref,
             *, g, bm, nph, cin, pp, taps):
    base = pl.program_id(1) * bm
    xw = x_ref[0, pl.ds(base, bm + 2 * g), :]
    mw = m_ref[pl.ds(base, bm + 2 * g), :]

    hs = []
    for p in range(nph):
        hp = jnp.dot(xw[:, p * cin:(p + 1) * cin], w1_ref[...],
                     preferred_element_type=jnp.float32)
        hs.append((jnp.maximum(hp * s1_ref[...] + t1_ref[...], 0.0)
                   * mw).astype(BF16))

    wide = jnp.concatenate(
        [hs[b][g + s:g + s + bm, :] for s, b in taps], axis=1)
    h2 = jnp.dot(wide, w2_ref[...], preferred_element_type=jnp.float32)
    h2 = jnp.maximum(h2 * s2_ref[...] + t2_ref[...], 0.0).astype(BF16)

    o3 = jnp.dot(h2, w3_ref[...], preferred_element_type=jnp.float32)
    o3 = o3 * s3_ref[...] + t3_ref[...]
    idn = jnp.dot(xw[g:g + bm, :cin], wd_ref[...],
                  preferred_element_type=jnp.float32)
    o3 = o3 + idn * sd_ref[...] + td_ref[...]
    o_ref[0] = jnp.maximum(o3, 0.0).astype(BF16)


def _pad_last(a, n):
    if a.shape[-1] == n:
        return a
    return jnp.pad(a, [(0, 0)] * (a.ndim - 1) + [(0, n - a.shape[-1])])


def _b0_fused(x, p, stride):
    B, H, W, C = x.shape
    if stride == 2:
        H, W = H // 2, W // 2
        x = x.reshape(B, H, 2, W, 2, C).transpose(0, 1, 3, 2, 4, 5)
        x = x.reshape(B, H, W, 4 * C)
        nph = 4
    else:
        nph = 1
    P0 = p["w1"].shape[1]
    C2 = p["w3"].shape[1]
    P = max(P0, 128)
    Cp = max(C, 128)
    x = _pad_last(x, nph * Cp) if C < 128 else x

    Wp = W + 2
    n = (H + 2) * Wp
    n_pad = _ru(n, 8)
    G = _ru(Wp + 1, 8)
    Rg = (B // 2) * (G + n_pad) + G
    T = -(-Rg // 1536)
    bm = _ru(-(-Rg // T), 8)
    Rgp = bm * T

    x0 = _fold_groups(x.astype(BF16), G, n_pad)
    x0 = jnp.pad(x0, ((0, 0), (0, Rgp + 2 * G - Rg), (0, 0)))
    mask = _row_mask(H, W, G, n_pad, B // 2)
    mask = jnp.pad(mask, ((0, Rgp + 2 * G - Rg), (0, 0)))

    w1 = _pad2d(p["w1"].astype(BF16), Cp, P)
    s1, t1 = _pad2d(p["s1"], 1, P), _pad2d(p["t1"], 1, P)
    w2 = p["w2"].reshape(3, 3, P0, P0).reshape(9, P0, P0)
    w2 = jnp.pad(w2, ((0, 0), (0, P - P0), (0, P - P0)))
    w2 = w2.reshape(9 * P, P).astype(BF16)
    s2, t2 = _pad2d(p["s2"], 1, P), _pad2d(p["t2"], 1, P)
    w3 = _pad2d(p["w3"].astype(BF16), P, C2)
    wd = _pad2d(p["wd"].astype(BF16), Cp, C2)

    if nph == 1:
        taps = [(dh * Wp + dw, 0) for dh in (-1, 0, 1) for dw in (-1, 0, 1)]
    else:
        def phm(k):
            return (1, -1) if k == -1 else (0, 0) if k == 0 else (1, 0)
        taps = []
        for kh in (-1, 0, 1):
            ph, di = phm(kh)
            for kw in (-1, 0, 1):
                pw, dj = phm(kw)
                taps.append((di * Wp + dj, ph * 2 + pw))

    Cx = x0.shape[-1]
    out = pl.pallas_call(
        functools.partial(_b0_body, g=G, bm=bm, nph=nph, cin=Cp, pp=P,
                          taps=taps),
        out_shape=jax.ShapeDtypeStruct((2, Rgp, C2), BF16),
        grid=(2, T),
        in_specs=[pl.BlockSpec((1, Rgp + 2 * G, Cx), lambda g, i: (g, 0, 0)),
                  pl.BlockSpec((Rgp + 2 * G, 1), lambda g, i: (0, 0)),
                  pl.BlockSpec((Cp, P), lambda g, i: (0, 0)),
                  pl.BlockSpec((1, P), lambda g, i: (0, 0)),
                  pl.BlockSpec((1, P), lambda g, i: (0, 0)),
                  pl.BlockSpec((9 * P, P), lambda g, i: (0, 0)),
                  pl.BlockSpec((1, P), lambda g, i: (0, 0)),
                  pl.BlockSpec((1, P), lambda g, i: (0, 0)),
                  pl.BlockSpec((P, C2), lambda g, i: (0, 0)),
                  pl.BlockSpec((1, C2), lambda g, i: (0, 0)),
                  pl.BlockSpec((1, C2), lambda g, i: (0, 0)),
                  pl.BlockSpec((Cp, C2), lambda g, i: (0, 0)),
                  pl.BlockSpec((1, C2), lambda g, i: (0, 0)),
                  pl.BlockSpec((1, C2), lambda g, i: (0, 0))],
        out_specs=pl.BlockSpec((1, bm, C2), lambda g, i: (g, i, 0)),
        compiler_params=pltpu.CompilerParams(
            dimension_semantics=("parallel", "parallel"),
            vmem_limit_bytes=64 * 1024 * 1024),
    )(x0, mask, w1, s1, t1, w2, s2, t2, w3, p["s3"], p["t3"],
      wd, p["sd"], p["td"])

    seg = G + n_pad
    body = out.reshape(2, Rgp, C2)[:, :(B // 2) * seg, :].reshape(B, seg, C2)
    body = body[:, :n, :].reshape(B, H + 2, Wp, C2)
    return body[:, 1:H + 1, 1:W + 1, :]


def _stem_body(x_ref, w_ref, s_ref, t_ref, o_ref, *, wrow, bm):
    base = pl.program_id(1) * bm
    win = x_ref[0, pl.ds(base, bm + 256), :]
    shifts = [dh * wrow + dw for dh in range(4) for dw in range(4)]
    wide = jnp.concatenate([win[s:s + bm, :] for s in shifts], axis=1)
    h = jnp.dot(wide, w_ref[...], preferred_element_type=jnp.float32)
    o_ref[0] = jnp.maximum(h * s_ref[...] + t_ref[...], 0.0).astype(BF16)


def _stem_conv(x_nchw, w, s, t):
    B = x_nchw.shape[0]
    x = x_nchw.astype(BF16)
    x = jnp.pad(x, ((0, 0), (0, 1), (3, 3), (3, 3)))
    x = x.reshape(B, 4, 67, 2, 67, 2).transpose(0, 2, 4, 3, 5, 1)
    x = x.reshape(B, 67, 67, 16)
    n, n_pad, tail, bm = 67 * 67, 4608, 256, 2304
    rows = (B // 2) * n_pad
    x = jnp.pad(x.reshape(B, n, 16), ((0, 0), (0, n_pad - n), (0, 0)))
    x = x.reshape(2, rows, 16)
    x = jnp.pad(x, ((0, 0), (0, tail), (0, 0)))

    wp = jnp.pad(w, ((0, 1), (0, 1), (0, 1), (0, 0)))
    wk = wp.reshape(4, 2, 4, 2, 4, 64).transpose(0, 2, 1, 3, 4, 5)
    wk = wk.reshape(256, 64).astype(BF16)
    wk = jnp.pad(wk, ((0, 0), (0, 64)))
    sp = _pad2d(s, 1, 128)
    tp = _pad2d(t, 1, 128)

    out = pl.pallas_call(
        functools.partial(_stem_body, wrow=67, bm=bm),
        out_shape=jax.ShapeDtypeStruct((2, rows, 128), BF16),
        grid=(2, rows // bm),
        in_specs=[pl.BlockSpec((1, rows + tail, 16), lambda g, i: (g, 0, 0)),
                  pl.BlockSpec((256, 128), lambda g, i: (0, 0)),
                  pl.BlockSpec((1, 128), lambda g, i: (0, 0)),
                  pl.BlockSpec((1, 128), lambda g, i: (0, 0))],
        out_specs=pl.BlockSpec((1, bm, 128), lambda g, i: (g, i, 0)),
        compiler_params=pltpu.CompilerParams(
            dimension_semantics=("parallel", "parallel"),
            vmem_limit_bytes=64 * 1024 * 1024),
    )(x, wk, sp, tp)

    out = out.reshape(B, n_pad, 128)[:, :n, :].reshape(B, 67, 67, 128)
    return out[:, :64, :64, :]


def _head_body(f_ref, m_ref, y_ref, wr_ref, br_ref, wu_ref, bu_ref,
               wfr_ref, wfu_ref, bf_ref, o_ref, *, seg, npix):
    acts = f_ref[...].astype(jnp.float32) * m_ref[...]
    g0 = [jnp.sum(acts[0, k * seg:(k + 1) * seg, :], axis=0) for k in range(4)]
    g1 = [jnp.sum(acts[1, k * seg:(k + 1) * seg, :], axis=0) for k in range(4)]
    pooled = jnp.stack(g0 + g1) * (1.0 / npix)
    r = jnp.dot(pooled.astype(BF16), wr_ref[...],
                preferred_element_type=jnp.float32) + br_ref[...]
    u = jnp.dot(y_ref[...], wu_ref[...],
                preferred_element_type=jnp.float32) + bu_ref[...]
    z = (jnp.dot(r.astype(BF16), wfr_ref[...],
                 preferred_element_type=jnp.float32)
         + jnp.dot(u.astype(BF16), wfu_ref[...],
                   preferred_element_type=jnp.float32)
         + bf_ref[...])
    o_ref[...] = z


def _head(folded, mask, seg, npix, y2, wr, br, wu, bu, wfr, wfu, bf):
    B = 8
    args = (folded, mask, y2, wr, br, wu, bu, wfr, wfu, bf)
    in_specs = [pl.BlockSpec(a.shape, lambda i, _nd=a.ndim: (0,) * _nd)
                for a in args]
    out = pl.pallas_call(
        functools.partial(_head_body, seg=seg, npix=npix),
        out_shape=jax.ShapeDtypeStruct((B, 128), jnp.float32),
        grid=(1,),
        in_specs=in_specs,
        out_specs=pl.BlockSpec((B, 128), lambda i: (0, 0)),
        compiler_params=pltpu.CompilerParams(
            vmem_limit_bytes=64 * 1024 * 1024),
    )(*args)
    return out[:, :50]


def _bottleneck0(x, p, stride):
    idn = _conv1x1(x, p["wd"], p["sd"], p["td"], stride=stride, relu=False)
    h = _conv1x1(x, p["w1"], p["s1"], p["t1"], relu=True)
    h = _conv_knxn(h, p["w2"], p["s2"], p["t2"],
                   stride=stride, padding=1, relu=True)
    return _conv1x1(h, p["w3"], p["s3"], p["t3"], relu=True, residual=idn)


def kernel(conv1_w, bn1_s, bn1_sh,
           l1_b0_w1, l1_b0_s1, l1_b0_sh1, l1_b0_w2, l1_b0_s2, l1_b0_sh2,
           l1_b0_w3, l1_b0_s3, l1_b0_sh3, l1_b0_wd, l1_b0_sd, l1_b0_shd,
           l1_rest_w1, l1_rest_s1, l1_rest_sh1, l1_rest_w2, l1_rest_s2,
           l1_rest_sh2, l1_rest_w3, l1_rest_s3, l1_rest_sh3,
           l2_b0_w1, l2_b0_s1, l2_b0_sh1, l2_b0_w2, l2_b0_s2, l2_b0_sh2,
           l2_b0_w3, l2_b0_s3, l2_b0_sh3, l2_b0_wd, l2_b0_sd, l2_b0_shd,
           l2_rest_w1, l2_rest_s1, l2_rest_sh1, l2_rest_w2, l2_rest_s2,
           l2_rest_sh2, l2_rest_w3, l2_rest_s3, l2_rest_sh3,
           l3_b0_w1, l3_b0_s1, l3_b0_sh1, l3_b0_w2, l3_b0_s2, l3_b0_sh2,
           l3_b0_w3, l3_b0_s3, l3_b0_sh3, l3_b0_wd, l3_b0_sd, l3_b0_shd,
           l3_rest_w1, l3_rest_s1, l3_rest_sh1, l3_rest_w2, l3_rest_s2,
           l3_rest_sh2, l3_rest_w3, l3_rest_s3, l3_rest_sh3,
           wr, br, wu, bu, wfr, wfu, bf,
           x_nchw, y):
    x = _stem_conv(x_nchw, conv1_w, bn1_s, bn1_sh)
    x = _maxpool_3x3_s2(x)

    layers = (
        (1, {"w1": l1_b0_w1, "s1": l1_b0_s1, "t1": l1_b0_sh1,
             "w2": l1_b0_w2, "s2": l1_b0_s2, "t2": l1_b0_sh2,
             "w3": l1_b0_w3, "s3": l1_b0_s3, "t3": l1_b0_sh3,
             "wd": l1_b0_wd, "sd": l1_b0_sd, "td": l1_b0_shd},
         (l1_rest_w1, l1_rest_s1, l1_rest_sh1, l1_rest_w2, l1_rest_s2,
          l1_rest_sh2, l1_rest_w3, l1_rest_s3, l1_rest_sh3)),
        (2, {"w1": l2_b0_w1, "s1": l2_b0_s1, "t1": l2_b0_sh1,
             "w2": l2_b0_w2, "s2": l2_b0_s2, "t2": l2_b0_sh2,
             "w3": l2_b0_w3, "s3": l2_b0_s3, "t3": l2_b0_sh3,
             "wd": l2_b0_wd, "sd": l2_b0_sd, "td": l2_b0_shd},
         (l2_rest_w1, l2_rest_s1, l2_rest_sh1, l2_rest_w2, l2_rest_s2,
          l2_rest_sh2, l2_rest_w3, l2_rest_s3, l2_rest_sh3)),
        (2, {"w1": l3_b0_w1, "s1": l3_b0_s1, "t1": l3_b0_sh1,
             "w2": l3_b0_w2, "s2": l3_b0_s2, "t2": l3_b0_sh2,
             "w3": l3_b0_w3, "s3": l3_b0_s3, "t3": l3_b0_sh3,
             "wd": l3_b0_wd, "sd": l3_b0_sd, "td": l3_b0_shd},
         (l3_rest_w1, l3_rest_s1, l3_rest_sh1, l3_rest_w2, l3_rest_s2,
          l3_rest_sh2, l3_rest_w3, l3_rest_s3, l3_rest_sh3)),
    )

    for li, (stride0, b0, rest) in enumerate(layers):
        x = _b0_fused(x, b0, stride0)
        x = _bottleneck_stack(x, *rest, keep_folded=(li == 2))

    folded, mask, seg, npix = x
    y2 = y.reshape(y.shape[0], -1).astype(BF16)
    y2 = jnp.pad(y2, ((0, 0), (0, 256 - y2.shape[1])))
    return _head(folded, mask, seg, npix, y2, wr, br, wu, bu, wfr, wfu, bf)
